# baseline (speedup 1.0000x reference)
import functools

import jax
import jax.numpy as jnp
from jax import lax
from jax.experimental import pallas as pl
from jax.experimental.pallas import tpu as pltpu
from jax.experimental.pallas import tpu_sc as plsc

N_NODES = 10000
N_EDGES = 320000
N_GRAPHS = 64
HID = 128
PADN = 10240
SLOPE = 0.01

_CH = 2000
_NCH = N_EDGES // _CH

_NC = 2
_NS = 16
_NW = _NC * _NS
_CPT = HID // _NW

@functools.cache
def _get_sc_edge():
    mesh = plsc.VectorSubcoreMesh(core_axis_name="c", subcore_axis_name="s",
                                  num_cores=_NC, num_subcores=_NS)
    return functools.partial(
        pl.kernel,
        mesh=mesh,
        compiler_params=pltpu.CompilerParams(needs_layout_passes=False),
        out_type=(
            jax.ShapeDtypeStruct((HID, PADN), jnp.float32),
            jax.ShapeDtypeStruct((_NC * N_EDGES,), jnp.float32),
            jax.ShapeDtypeStruct((_NC * _NS * PADN,), jnp.float32),
            jax.ShapeDtypeStruct((_NC * PADN,), jnp.float32),
        ),
        scratch_types=[
            pltpu.VMEM((_CPT, PADN), jnp.float32),
            pltpu.VMEM((_CPT, PADN), jnp.float32),
            pltpu.VMEM((PADN,), jnp.float32),
            pltpu.VMEM((PADN,), jnp.float32),
            pltpu.VMEM((PADN,), jnp.float32),
            pltpu.VMEM((2, 128), jnp.float32),
            pltpu.VMEM((_CH,), jnp.int32),
            pltpu.VMEM((_CH,), jnp.int32),
            pltpu.VMEM((_CH,), jnp.int32),
            pltpu.VMEM((_CH,), jnp.int32),
            pltpu.VMEM((_CH,), jnp.float32),
            pltpu.VMEM((_CH,), jnp.float32),
            pltpu.VMEM((_NS * 160,), jnp.float32),
            pltpu.VMEM((PADN // _NS,), jnp.float32),
            pltpu.SemaphoreType.DMA,
            pltpu.SemaphoreType.DMA,
            pltpu.SemaphoreType.DMA,
            pltpu.SemaphoreType.DMA,
            pltpu.SemaphoreType.DMA,
            pltpu.SemaphoreType.DMA,
        ],
    )(_sc_edge_body)


_EPT = N_EDGES // _NS
_NCH1 = _EPT // _CH
_NSL = PADN // _NS


def _sc_edge_body(val_hbm, asrc_hbm, adst_hbm, mx_hbm, src_hbm, dst_hbm,
                  out_hbm, ex_hbm, part_hbm, sred_hbm, xc, hc, av, bv, sv,
                  mv, sb0, db0, sb1, db1, eb0, eb1, rb, accv,
                  sem_s0, sem_d0, sem_s1, sem_d1, sem_e0, sem_e1):
    cid = lax.axis_index("c")
    sid = lax.axis_index("s")
    wid = sid * _NC + cid
    base = wid * _CPT
    bufs = ((sb0, db0, sem_s0, sem_d0, eb0, sem_e0),
            (sb1, db1, sem_s1, sem_d1, eb1, sem_e1))

    def _start_sd(b, k):
        sb, db, ss, sd, _, _ = bufs[b]
        pltpu.make_async_copy(src_hbm.at[pl.ds(k * _CH, _CH)], sb, ss).start()
        pltpu.make_async_copy(dst_hbm.at[pl.ds(k * _CH, _CH)], db, sd).start()

    def _wait_sd(b, k):
        sb, db, ss, sd, _, _ = bufs[b]
        pltpu.make_async_copy(src_hbm.at[pl.ds(k * _CH, _CH)], sb, ss).wait()
        pltpu.make_async_copy(dst_hbm.at[pl.ds(k * _CH, _CH)], db, sd).wait()

    pltpu.sync_copy(val_hbm.at[pl.ds(base, _CPT)], xc)
    pltpu.sync_copy(asrc_hbm, av)
    pltpu.sync_copy(adst_hbm, bv)
    pltpu.sync_copy(mx_hbm, mv)
    cvec = jnp.maximum(mv[0, pl.ds(0, 16)] + mv[1, pl.ds(0, 16)], 0.0)

    zero16 = jnp.zeros((16,), jnp.float32)

    @plsc.parallel_loop(0, PADN // 16, unroll=8)
    def _zs(i):
        sv[pl.ds(i * 16, 16)] = zero16

    @plsc.parallel_loop(0, PADN // 16, unroll=8)
    def _zh(i):
        for c in range(_CPT):
            hc[c, pl.ds(i * 16, 16)] = zero16

    ebase = sid * _EPT

    exbase = cid * N_EDGES

    def _ex_out(b, k):
        eb, se = bufs[b][4], bufs[b][5]
        return pltpu.make_async_copy(
            eb, ex_hbm.at[pl.ds(exbase + ebase + k * _CH, _CH)], se)

    def _start1(b, k):
        sb, db, ss, sd, _, _ = bufs[b]
        pltpu.make_async_copy(
            src_hbm.at[pl.ds(ebase + k * _CH, _CH)], sb, ss).start()
        pltpu.make_async_copy(
            dst_hbm.at[pl.ds(ebase + k * _CH, _CH)], db, sd).start()

    def _wait1(b, k):
        sb, db, ss, sd, _, _ = bufs[b]
        pltpu.make_async_copy(
            src_hbm.at[pl.ds(ebase + k * _CH, _CH)], sb, ss).wait()
        pltpu.make_async_copy(
            dst_hbm.at[pl.ds(ebase + k * _CH, _CH)], db, sd).wait()

    _start1(0, 0)
    for k in range(_NCH1):
        b = k % 2
        if k + 1 < _NCH1:
            _start1(1 - b, k + 1)
        _wait1(b, k)
        if k >= 2:
            _ex_out(b, k - 2).wait()
        sb, db, eb = bufs[b][0], bufs[b][1], bufs[b][4]

        @plsc.parallel_loop(0, _CH // 16, unroll=8)
        def _g1(i):
            svv = sb[pl.ds(i * 16, 16)]
            dvv = db[pl.ds(i * 16, 16)]
            t = plsc.load_gather(av, [svv]) + plsc.load_gather(bv, [dvv])
            al = jnp.maximum(t, t * SLOPE)
            ex = jnp.exp(al - cvec)
            eb[pl.ds(i * 16, 16)] = ex
            plsc.addupdate_scatter(sv, [dvv], ex)
        _ex_out(b, k).start()
    for k in (_NCH1 - 2, _NCH1 - 1):
        _ex_out(k % 2, k).wait()

    pbase = cid * _NS * PADN
    nbase = sid * _NSL
    pltpu.sync_copy(sv, part_hbm.at[pl.ds(pbase + sid * PADN, PADN)])
    plsc.subcore_barrier()

    for p in range(_NSL // 160):
        for j in range(_NS):
            pltpu.make_async_copy(
                part_hbm.at[pl.ds(pbase + j * PADN + nbase + p * 160, 160)],
                rb.at[pl.ds(j * 160, 160)], sem_e0).start()
        for j in range(_NS):
            pltpu.make_async_copy(
                part_hbm.at[pl.ds(pbase + j * PADN + nbase + p * 160, 160)],
                rb.at[pl.ds(j * 160, 160)], sem_e0).wait()

        @plsc.parallel_loop(0, 10, unroll=2)
        def _r(i):
            acc = rb[pl.ds(i * 16, 16)]
            for j in range(1, _NS):
                acc = acc + rb[pl.ds(j * 160 + i * 16, 16)]
            accv[pl.ds(p * 160 + i * 16, 16)] = acc
    pltpu.sync_copy(accv, sred_hbm.at[pl.ds(cid * PADN + nbase, _NSL)])
    plsc.subcore_barrier()
    pltpu.sync_copy(sred_hbm.at[pl.ds(cid * PADN, PADN)], sv)

    def _start2(b, k):
        _start_sd(b, k)
        eb, se = bufs[b][4], bufs[b][5]
        pltpu.make_async_copy(
            ex_hbm.at[pl.ds(exbase + k * _CH, _CH)], eb, se).start()

    def _wait2(b, k):
        _wait_sd(b, k)
        eb, se = bufs[b][4], bufs[b][5]
        pltpu.make_async_copy(
            ex_hbm.at[pl.ds(exbase + k * _CH, _CH)], eb, se).wait()

    def _proc_h(sb, db, eb):
        @plsc.parallel_loop(0, _CH // 16, unroll=6)
        def _g(i):
            svv = sb[pl.ds(i * 16, 16)]
            dvv = db[pl.ds(i * 16, 16)]
            ex = eb[pl.ds(i * 16, 16)]
            w = ex / plsc.load_gather(sv, [dvv])
            for c in range(_CPT):
                cv16 = jnp.full((16,), c, jnp.int32)
                xvv = plsc.load_gather(xc, [cv16, svv])
                plsc.addupdate_scatter(hc, [cv16, dvv], xvv * w)

    _start2(0, 0)

    def _pair(k2, _):
        c0 = 2 * k2
        _start2(1, c0 + 1)
        _wait2(0, c0)
        _proc_h(bufs[0][0], bufs[0][1], bufs[0][4])
        nxt = jnp.minimum(c0 + 2, _NCH - 1)
        _start2(0, nxt)
        _wait2(1, c0 + 1)
        _proc_h(bufs[1][0], bufs[1][1], bufs[1][4])
        return 0
    lax.fori_loop(0, _NCH // 2, _pair, 0)
    _wait2(0, _NCH - 1)

    pltpu.sync_copy(hc, out_hbm.at[pl.ds(base, _CPT)])


def _mm(a, b):
    return jnp.dot(a, b, preferred_element_type=jnp.float32)


def _leaky(v):
    return jnp.maximum(v, v * SLOPE)


def _elu(v):
    vn = jnp.minimum(v, 0.0)
    u = jnp.exp(vn)
    em1 = jnp.where(u == 1.0, vn, (u - 1.0) * vn / jnp.log(u))
    return jnp.where(v > 0, v, em1)


def _gru_t(hin, xprev, wit, bib, wht, bhb):
    gi = _mm(wit, hin) + bib[:, :1]
    gh = _mm(wht, xprev) + bhb[:, :1]
    r = jax.nn.sigmoid(gi[0:HID] + gh[0:HID])
    z = jax.nn.sigmoid(gi[HID:2 * HID] + gh[HID:2 * HID])
    n = jnp.tanh(gi[2 * HID:] + r * gh[2 * HID:])
    return (1.0 - z) * n + z * xprev


def _prep_body(xt_ref, w1t_ref, b1_ref, gw1at_ref, w1b_ref, gw2t_ref,
               attl_ref, attr_ref,
               x0_ref, val_ref, asrc_ref, adst_ref, mx_ref):
    x0 = _leaky(_mm(w1t_ref[...], xt_ref[...]) + b1_ref[:, :1])
    x0_ref[...] = x0
    ht = _leaky(_mm(gw1at_ref[...], x0) + w1b_ref[:, :1])
    val_ref[...] = _mm(gw2t_ref[...], x0)
    asrc = _mm(attl_ref[...], ht)
    adst = _mm(attr_ref[...], x0)
    asrc_ref[...] = asrc
    adst_ref[...] = adst
    m0 = jnp.broadcast_to(jnp.max(asrc), (1, 128))
    m1 = jnp.broadcast_to(jnp.max(adst), (1, 128))
    mx_ref[...] = jnp.concatenate([m0, m1], axis=0)


def _build_tc_prep(interpret=False):
    return pl.pallas_call(
        _prep_body,
        out_shape=(
            jax.ShapeDtypeStruct((HID, PADN), jnp.float32),
            jax.ShapeDtypeStruct((HID, PADN), jnp.float32),
            jax.ShapeDtypeStruct((1, PADN), jnp.float32),
            jax.ShapeDtypeStruct((1, PADN), jnp.float32),
            jax.ShapeDtypeStruct((2, 128), jnp.float32),
        ),
        interpret=interpret,
    )


_tc_prep = _build_tc_prep()


_BW = PADN // 4


def _layer_body(agg_ref, xprev_ref, biasc_ref, wit_ref, bib_ref, wht_ref,
                bhb_ref, wlt_ref, ats_ref, atd_ref,
                xn_ref, xl_ref, asrc_ref, adst_ref, mx_ref):
    hh = _elu(agg_ref[...] + biasc_ref[:, :1])
    xn = jnp.maximum(
        _gru_t(hh, xprev_ref[...], wit_ref[...], bib_ref[...],
               wht_ref[...], bhb_ref[...]), 0.0)
    xn_ref[...] = xn
    xl = _mm(wlt_ref[...], xn)
    xl_ref[...] = xl
    asrc = _mm(ats_ref[...], xl)
    adst = _mm(atd_ref[...], xl)
    asrc_ref[...] = asrc
    adst_ref[...] = adst
    cur = jnp.concatenate(
        [jnp.broadcast_to(jnp.max(asrc), (1, 128)),
         jnp.broadcast_to(jnp.max(adst), (1, 128))], axis=0)

    @pl.when(pl.program_id(0) == 0)
    def _():
        mx_ref[...] = cur

    @pl.when(pl.program_id(0) > 0)
    def _():
        mx_ref[...] = jnp.maximum(mx_ref[...], cur)


def _build_tc_layer(interpret=False):
    return pl.pallas_call(
        _layer_body,
        grid=(4,),
        interpret=interpret,
        in_specs=[
            pl.BlockSpec((HID, _BW), lambda i: (0, i)),
            pl.BlockSpec((HID, _BW), lambda i: (0, i)),
            pl.BlockSpec((HID, 128), lambda i: (0, 0)),
            pl.BlockSpec((3 * HID, HID), lambda i: (0, 0)),
            pl.BlockSpec((3 * HID, 128), lambda i: (0, 0)),
            pl.BlockSpec((3 * HID, HID), lambda i: (0, 0)),
            pl.BlockSpec((3 * HID, 128), lambda i: (0, 0)),
            pl.BlockSpec((HID, HID), lambda i: (0, 0)),
            pl.BlockSpec((1, HID), lambda i: (0, 0)),
            pl.BlockSpec((1, HID), lambda i: (0, 0)),
        ],
        out_specs=(
            pl.BlockSpec((HID, _BW), lambda i: (0, i)),
            pl.BlockSpec((HID, _BW), lambda i: (0, i)),
            pl.BlockSpec((1, _BW), lambda i: (0, i)),
            pl.BlockSpec((1, _BW), lambda i: (0, i)),
            pl.BlockSpec((2, 128), lambda i: (0, 0)),
        ),
        out_shape=(
            jax.ShapeDtypeStruct((HID, PADN), jnp.float32),
            jax.ShapeDtypeStruct((HID, PADN), jnp.float32),
            jax.ShapeDtypeStruct((1, PADN), jnp.float32),
            jax.ShapeDtypeStruct((1, PADN), jnp.float32),
            jax.ShapeDtypeStruct((2, 128), jnp.float32),
        ),
    )


_tc_layer = _build_tc_layer()


def _pool_body(xn_ref, brow_ref, bcol_ref, molwt_ref, atts_ref, attd_ref,
               molb_ref, mwit_ref, mbib_ref, mwht_ref, mbhb_ref,
               w2t_ref, b2_ref, pw1a_ref, pw1b_ref, pb1_ref, pw2_ref,
               pb2_ref, out_ref):
    xn = xn_ref[...]
    brow = brow_ref[...]
    bcol = bcol_ref[...]
    gid_r = lax.broadcasted_iota(jnp.int32, (N_GRAPHS, PADN), 0)
    ott = gid_r == brow
    ottf = ott.astype(jnp.float32)
    gid_c = lax.broadcasted_iota(jnp.int32, (PADN, N_GRAPHS), 1)
    otf = (gid_c == bcol).astype(jnp.float32)

    out = jnp.maximum(_mm(xn, otf), 0.0)

    for _ in range(2):
        xs = _mm(molwt_ref[...], xn)
        xd = _mm(molwt_ref[...], out)
        pern = _mm(_mm(attd_ref[...], xd), ottf)
        a = _leaky(_mm(atts_ref[...], xs) + pern)
        am = jnp.where(ott, jnp.broadcast_to(a, (N_GRAPHS, PADN)), -1e30)
        amax = jnp.max(am, axis=1, keepdims=True)
        amax = jnp.where(amax > -1e29, amax, 0.0)
        apern = _mm(jnp.reshape(amax, (1, N_GRAPHS)), ottf)
        ex = jnp.exp(a - apern)
        s = _mm(ex, otf)
        spern = _mm(s, ottf)
        w = ex / (spern + 1e-16)
        hg = _mm(xs * w, otf) + molb_ref[:, :1]
        hg = _elu(hg)
        out = jnp.maximum(
            _gru_t(hg, out, mwit_ref[...], mbib_ref[...],
                   mwht_ref[...], mbhb_ref[...]), 0.0)

    g = _mm(w2t_ref[...], out) + b2_ref[...]
    gr = jnp.maximum(g[0:1] * pw1a_ref[...] + g[1:2] * pw1b_ref[...]
                     + pb1_ref[...], 0.0)
    out_ref[...] = gr * pw2_ref[...] + pb2_ref[...]


def _build_tc_pool(interpret=False):
    return pl.pallas_call(
        _pool_body,
        out_shape=jax.ShapeDtypeStruct((1, N_GRAPHS), jnp.float32),
        interpret=interpret,
    )


_tc_pool = _build_tc_pool()


def kernel(x, edge_index, batch, edge_attr, W_lin1, b_lin1, gate_W1, gate_W2,
           gate_att_l, gate_att_r, gate_bias, gru0_Wi, gru0_Wh, gru0_bi,
           gru0_bh, conv_W, conv_att_src, conv_att_dst, conv_bias, grus_Wi,
           grus_Wh, grus_bi, grus_bh, mol_W, mol_att_src, mol_att_dst,
           mol_bias, molgru_Wi, molgru_Wh, molgru_bi, molgru_bh, W_lin2,
           b_lin2, pm_W1, pm_b1, pm_W2, pm_b2):
    f32 = jnp.float32
    src = edge_index[0].astype(jnp.int32)
    dst = edge_index[1].astype(jnp.int32)
    pad = PADN - N_NODES
    xT = jnp.pad(x, ((0, pad), (0, 0))).T
    batch_p = jnp.pad(batch.astype(jnp.int32), (0, pad),
                      constant_values=N_GRAPHS)

    def bcast(v, rows):
        return jnp.broadcast_to(v.astype(f32)[:, None], (rows, 128))

    x0T, valT, asrc, adst, mx = _tc_prep(
        xT, W_lin1.T, bcast(b_lin1, HID), gate_W1[:HID].T,
        bcast(gate_W1[HID], HID), gate_W2.T, gate_att_l[None, :],
        gate_att_r[None, :])
    sc_edge = _get_sc_edge()
    aggT = sc_edge(valT, asrc.reshape(PADN), adst.reshape(PADN), mx,
                   src, dst)[0]

    xcur = x0T
    L = conv_W.shape[0]
    for l in range(L + 1):
        if l == 0:
            biasc = bcast(gate_bias, HID)
            wit, wht = gru0_Wi.T, gru0_Wh.T
            bib, bhb = bcast(gru0_bi, 3 * HID), bcast(gru0_bh, 3 * HID)
        else:
            biasc = bcast(conv_bias[l - 1], HID)
            wit, wht = grus_Wi[l - 1].T, grus_Wh[l - 1].T
            bib = bcast(grus_bi[l - 1], 3 * HID)
            bhb = bcast(grus_bh[l - 1], 3 * HID)
        li = min(l, L - 1)
        xcur, xlT, asrc, adst, mx = _tc_layer(
            aggT, xcur, biasc, wit, bib, wht, bhb, conv_W[li].T,
            conv_att_src[li][None, :], conv_att_dst[li][None, :])
        if l < L:
            aggT = sc_edge(xlT, asrc.reshape(PADN), adst.reshape(PADN),
                           mx, src, dst)[0]

    def brow64(v):
        return jnp.broadcast_to(v.astype(f32)[None, None], (1, N_GRAPHS))

    g = _tc_pool(
        xcur, batch_p[None, :], batch_p[:, None], mol_W.T,
        mol_att_src[None, :], mol_att_dst[None, :], bcast(mol_bias, HID),
        molgru_Wi.T, bcast(molgru_bi, 3 * HID), molgru_Wh.T,
        bcast(molgru_bh, 3 * HID), W_lin2.T,
        jnp.broadcast_to(b_lin2.astype(f32)[:, None], (2, N_GRAPHS)),
        brow64(pm_W1[0, 0]), brow64(pm_W1[1, 0]), brow64(pm_b1[0]),
        brow64(pm_W2[0, 0]), brow64(pm_b2[0]))
    return g.reshape(N_GRAPHS, 1)


# speedup vs baseline: 1.0581x; 1.0581x over previous
import functools

import jax
import jax.numpy as jnp
from jax import lax
from jax.experimental import pallas as pl
from jax.experimental.pallas import tpu as pltpu
from jax.experimental.pallas import tpu_sc as plsc

N_NODES = 10000
N_EDGES = 320000
N_GRAPHS = 64
HID = 128
PADN = 10240
SLOPE = 0.01

_CH = 2000
_NCH = N_EDGES // _CH

_NC = 2
_NS = 16
_NW = _NC * _NS
_CPT = HID // _NW

@functools.cache
def _get_sc_edge():
    mesh = plsc.VectorSubcoreMesh(core_axis_name="c", subcore_axis_name="s",
                                  num_cores=_NC, num_subcores=_NS)
    return functools.partial(
        pl.kernel,
        mesh=mesh,
        compiler_params=pltpu.CompilerParams(needs_layout_passes=False),
        out_type=(
            jax.ShapeDtypeStruct((HID, PADN), jnp.float32),
            jax.ShapeDtypeStruct((_NC * N_EDGES,), jnp.float32),
            jax.ShapeDtypeStruct((_NC * _NS * PADN,), jnp.float32),
            jax.ShapeDtypeStruct((_NC * PADN,), jnp.float32),
        ),
        scratch_types=[
            pltpu.VMEM((_CPT, PADN), jnp.float32),
            pltpu.VMEM((_CPT, PADN), jnp.float32),
            pltpu.VMEM((PADN,), jnp.float32),
            pltpu.VMEM((PADN,), jnp.float32),
            pltpu.VMEM((PADN,), jnp.float32),
            pltpu.VMEM((2, 128), jnp.float32),
            pltpu.VMEM((_CH,), jnp.int32),
            pltpu.VMEM((_CH,), jnp.int32),
            pltpu.VMEM((_CH,), jnp.int32),
            pltpu.VMEM((_CH,), jnp.int32),
            pltpu.VMEM((_CH,), jnp.float32),
            pltpu.VMEM((_CH,), jnp.float32),
            pltpu.VMEM((_NS * 160,), jnp.float32),
            pltpu.VMEM((PADN // _NS,), jnp.float32),
            pltpu.SemaphoreType.DMA,
            pltpu.SemaphoreType.DMA,
            pltpu.SemaphoreType.DMA,
            pltpu.SemaphoreType.DMA,
            pltpu.SemaphoreType.DMA,
            pltpu.SemaphoreType.DMA,
        ],
    )(_sc_edge_body)


_EPT = N_EDGES // _NS
_NCH1 = _EPT // _CH
_NSL = PADN // _NS


def _sc_edge_body(val_hbm, asrc_hbm, adst_hbm, mx_hbm, src_hbm, dst_hbm,
                  out_hbm, ex_hbm, part_hbm, sred_hbm, xc, hc, av, bv, sv,
                  mv, sb0, db0, sb1, db1, eb0, eb1, rb, accv,
                  sem_s0, sem_d0, sem_s1, sem_d1, sem_e0, sem_e1):
    cid = lax.axis_index("c")
    sid = lax.axis_index("s")
    wid = sid * _NC + cid
    base = wid * _CPT
    bufs = ((sb0, db0, sem_s0, sem_d0, eb0, sem_e0),
            (sb1, db1, sem_s1, sem_d1, eb1, sem_e1))

    def _start_sd(b, k):
        sb, db, ss, sd, _, _ = bufs[b]
        pltpu.make_async_copy(src_hbm.at[pl.ds(k * _CH, _CH)], sb, ss).start()
        pltpu.make_async_copy(dst_hbm.at[pl.ds(k * _CH, _CH)], db, sd).start()

    def _wait_sd(b, k):
        sb, db, ss, sd, _, _ = bufs[b]
        pltpu.make_async_copy(src_hbm.at[pl.ds(k * _CH, _CH)], sb, ss).wait()
        pltpu.make_async_copy(dst_hbm.at[pl.ds(k * _CH, _CH)], db, sd).wait()

    pltpu.sync_copy(val_hbm.at[pl.ds(base, _CPT)], xc)
    pltpu.sync_copy(asrc_hbm, av)
    pltpu.sync_copy(adst_hbm, bv)
    pltpu.sync_copy(mx_hbm, mv)
    cvec = jnp.maximum(mv[0, pl.ds(0, 16)] + mv[1, pl.ds(0, 16)], 0.0)

    zero16 = jnp.zeros((16,), jnp.float32)

    @plsc.parallel_loop(0, PADN // 16, unroll=8)
    def _zs(i):
        sv[pl.ds(i * 16, 16)] = zero16

    @plsc.parallel_loop(0, PADN // 16, unroll=8)
    def _zh(i):
        for c in range(_CPT):
            hc[c, pl.ds(i * 16, 16)] = zero16

    ebase = sid * _EPT

    exbase = cid * N_EDGES

    def _ex_out(b, k):
        eb, se = bufs[b][4], bufs[b][5]
        return pltpu.make_async_copy(
            eb, ex_hbm.at[pl.ds(exbase + ebase + k * _CH, _CH)], se)

    def _start1(b, k):
        sb, db, ss, sd, _, _ = bufs[b]
        pltpu.make_async_copy(
            src_hbm.at[pl.ds(ebase + k * _CH, _CH)], sb, ss).start()
        pltpu.make_async_copy(
            dst_hbm.at[pl.ds(ebase + k * _CH, _CH)], db, sd).start()

    def _wait1(b, k):
        sb, db, ss, sd, _, _ = bufs[b]
        pltpu.make_async_copy(
            src_hbm.at[pl.ds(ebase + k * _CH, _CH)], sb, ss).wait()
        pltpu.make_async_copy(
            dst_hbm.at[pl.ds(ebase + k * _CH, _CH)], db, sd).wait()

    _start1(0, 0)
    for k in range(_NCH1):
        b = k % 2
        if k + 1 < _NCH1:
            _start1(1 - b, k + 1)
        _wait1(b, k)
        if k >= 2:
            _ex_out(b, k - 2).wait()
        sb, db, eb = bufs[b][0], bufs[b][1], bufs[b][4]

        @plsc.parallel_loop(0, _CH // 16, unroll=8)
        def _g1(i):
            svv = sb[pl.ds(i * 16, 16)]
            dvv = db[pl.ds(i * 16, 16)]
            t = plsc.load_gather(av, [svv]) + plsc.load_gather(bv, [dvv])
            al = jnp.maximum(t, t * SLOPE)
            ex = jnp.exp(al - cvec)
            eb[pl.ds(i * 16, 16)] = ex
            plsc.addupdate_scatter(sv, [dvv], ex)
        _ex_out(b, k).start()
    for k in (_NCH1 - 2, _NCH1 - 1):
        _ex_out(k % 2, k).wait()

    pbase = cid * _NS * PADN
    nbase = sid * _NSL
    pltpu.sync_copy(sv, part_hbm.at[pl.ds(pbase + sid * PADN, PADN)])
    plsc.subcore_barrier()

    for p in range(_NSL // 160):
        for j in range(_NS):
            pltpu.make_async_copy(
                part_hbm.at[pl.ds(pbase + j * PADN + nbase + p * 160, 160)],
                rb.at[pl.ds(j * 160, 160)], sem_e0).start()
        for j in range(_NS):
            pltpu.make_async_copy(
                part_hbm.at[pl.ds(pbase + j * PADN + nbase + p * 160, 160)],
                rb.at[pl.ds(j * 160, 160)], sem_e0).wait()

        @plsc.parallel_loop(0, 10, unroll=2)
        def _r(i):
            acc = rb[pl.ds(i * 16, 16)]
            for j in range(1, _NS):
                acc = acc + rb[pl.ds(j * 160 + i * 16, 16)]
            accv[pl.ds(p * 160 + i * 16, 16)] = acc
    pltpu.sync_copy(accv, sred_hbm.at[pl.ds(cid * PADN + nbase, _NSL)])
    plsc.subcore_barrier()
    pltpu.sync_copy(sred_hbm.at[pl.ds(cid * PADN, PADN)], sv)

    def _start2(b, k):
        _start_sd(b, k)
        eb, se = bufs[b][4], bufs[b][5]
        pltpu.make_async_copy(
            ex_hbm.at[pl.ds(exbase + k * _CH, _CH)], eb, se).start()

    def _wait2(b, k):
        _wait_sd(b, k)
        eb, se = bufs[b][4], bufs[b][5]
        pltpu.make_async_copy(
            ex_hbm.at[pl.ds(exbase + k * _CH, _CH)], eb, se).wait()

    def _proc_h(sb, db, eb):
        @plsc.parallel_loop(0, _CH // 16, unroll=4)
        def _g(i):
            svv = sb[pl.ds(i * 16, 16)]
            dvv = db[pl.ds(i * 16, 16)]
            ex = eb[pl.ds(i * 16, 16)]
            w = ex / plsc.load_gather(sv, [dvv])
            for c in range(_CPT):
                cv16 = jnp.full((16,), c, jnp.int32)
                xvv = plsc.load_gather(xc, [cv16, svv])
                plsc.addupdate_scatter(hc, [cv16, dvv], xvv * w)

    _start2(0, 0)

    def _pair(k2, _):
        c0 = 2 * k2
        _start2(1, c0 + 1)
        _wait2(0, c0)
        _proc_h(bufs[0][0], bufs[0][1], bufs[0][4])
        nxt = jnp.minimum(c0 + 2, _NCH - 1)
        _start2(0, nxt)
        _wait2(1, c0 + 1)
        _proc_h(bufs[1][0], bufs[1][1], bufs[1][4])
        return 0
    lax.fori_loop(0, _NCH // 2, _pair, 0)
    _wait2(0, _NCH - 1)

    pltpu.sync_copy(hc, out_hbm.at[pl.ds(base, _CPT)])


def _mm(a, b):
    return jnp.dot(a, b, preferred_element_type=jnp.float32)


def _leaky(v):
    return jnp.maximum(v, v * SLOPE)


def _elu(v):
    vn = jnp.minimum(v, 0.0)
    u = jnp.exp(vn)
    em1 = jnp.where(u == 1.0, vn, (u - 1.0) * vn / jnp.log(u))
    return jnp.where(v > 0, v, em1)


def _gru_t(hin, xprev, wit, bib, wht, bhb):
    gi = _mm(wit, hin) + bib[:, :1]
    gh = _mm(wht, xprev) + bhb[:, :1]
    r = jax.nn.sigmoid(gi[0:HID] + gh[0:HID])
    z = jax.nn.sigmoid(gi[HID:2 * HID] + gh[HID:2 * HID])
    n = jnp.tanh(gi[2 * HID:] + r * gh[2 * HID:])
    return (1.0 - z) * n + z * xprev


def _prep_body(xt_ref, w1t_ref, b1_ref, gw1at_ref, w1b_ref, gw2t_ref,
               attl_ref, attr_ref,
               x0_ref, val_ref, asrc_ref, adst_ref, mx_ref):
    x0 = _leaky(_mm(w1t_ref[...], xt_ref[...]) + b1_ref[:, :1])
    x0_ref[...] = x0
    ht = _leaky(_mm(gw1at_ref[...], x0) + w1b_ref[:, :1])
    val_ref[...] = _mm(gw2t_ref[...], x0)
    asrc = _mm(attl_ref[...], ht)
    adst = _mm(attr_ref[...], x0)
    asrc_ref[...] = asrc
    adst_ref[...] = adst
    m0 = jnp.broadcast_to(jnp.max(asrc), (1, 128))
    m1 = jnp.broadcast_to(jnp.max(adst), (1, 128))
    mx_ref[...] = jnp.concatenate([m0, m1], axis=0)


def _build_tc_prep(interpret=False):
    return pl.pallas_call(
        _prep_body,
        out_shape=(
            jax.ShapeDtypeStruct((HID, PADN), jnp.float32),
            jax.ShapeDtypeStruct((HID, PADN), jnp.float32),
            jax.ShapeDtypeStruct((1, PADN), jnp.float32),
            jax.ShapeDtypeStruct((1, PADN), jnp.float32),
            jax.ShapeDtypeStruct((2, 128), jnp.float32),
        ),
        interpret=interpret,
    )


_tc_prep = _build_tc_prep()


_BW = PADN // 4


def _layer_body(agg_ref, xprev_ref, biasc_ref, wit_ref, bib_ref, wht_ref,
                bhb_ref, wlt_ref, ats_ref, atd_ref,
                xn_ref, xl_ref, asrc_ref, adst_ref, mx_ref):
    hh = _elu(agg_ref[...] + biasc_ref[:, :1])
    xn = jnp.maximum(
        _gru_t(hh, xprev_ref[...], wit_ref[...], bib_ref[...],
               wht_ref[...], bhb_ref[...]), 0.0)
    xn_ref[...] = xn
    xl = _mm(wlt_ref[...], xn)
    xl_ref[...] = xl
    asrc = _mm(ats_ref[...], xl)
    adst = _mm(atd_ref[...], xl)
    asrc_ref[...] = asrc
    adst_ref[...] = adst
    cur = jnp.concatenate(
        [jnp.broadcast_to(jnp.max(asrc), (1, 128)),
         jnp.broadcast_to(jnp.max(adst), (1, 128))], axis=0)

    @pl.when(pl.program_id(0) == 0)
    def _():
        mx_ref[...] = cur

    @pl.when(pl.program_id(0) > 0)
    def _():
        mx_ref[...] = jnp.maximum(mx_ref[...], cur)


def _build_tc_layer(interpret=False):
    return pl.pallas_call(
        _layer_body,
        grid=(4,),
        interpret=interpret,
        in_specs=[
            pl.BlockSpec((HID, _BW), lambda i: (0, i)),
            pl.BlockSpec((HID, _BW), lambda i: (0, i)),
            pl.BlockSpec((HID, 128), lambda i: (0, 0)),
            pl.BlockSpec((3 * HID, HID), lambda i: (0, 0)),
            pl.BlockSpec((3 * HID, 128), lambda i: (0, 0)),
            pl.BlockSpec((3 * HID, HID), lambda i: (0, 0)),
            pl.BlockSpec((3 * HID, 128), lambda i: (0, 0)),
            pl.BlockSpec((HID, HID), lambda i: (0, 0)),
            pl.BlockSpec((1, HID), lambda i: (0, 0)),
            pl.BlockSpec((1, HID), lambda i: (0, 0)),
        ],
        out_specs=(
            pl.BlockSpec((HID, _BW), lambda i: (0, i)),
            pl.BlockSpec((HID, _BW), lambda i: (0, i)),
            pl.BlockSpec((1, _BW), lambda i: (0, i)),
            pl.BlockSpec((1, _BW), lambda i: (0, i)),
            pl.BlockSpec((2, 128), lambda i: (0, 0)),
        ),
        out_shape=(
            jax.ShapeDtypeStruct((HID, PADN), jnp.float32),
            jax.ShapeDtypeStruct((HID, PADN), jnp.float32),
            jax.ShapeDtypeStruct((1, PADN), jnp.float32),
            jax.ShapeDtypeStruct((1, PADN), jnp.float32),
            jax.ShapeDtypeStruct((2, 128), jnp.float32),
        ),
    )


_tc_layer = _build_tc_layer()


def _pool_body(xn_ref, brow_ref, bcol_ref, molwt_ref, atts_ref, attd_ref,
               molb_ref, mwit_ref, mbib_ref, mwht_ref, mbhb_ref,
               w2t_ref, b2_ref, pw1a_ref, pw1b_ref, pb1_ref, pw2_ref,
               pb2_ref, out_ref):
    xn = xn_ref[...]
    brow = brow_ref[...]
    bcol = bcol_ref[...]
    gid_r = lax.broadcasted_iota(jnp.int32, (N_GRAPHS, PADN), 0)
    ott = gid_r == brow
    ottf = ott.astype(jnp.float32)
    gid_c = lax.broadcasted_iota(jnp.int32, (PADN, N_GRAPHS), 1)
    otf = (gid_c == bcol).astype(jnp.float32)

    out = jnp.maximum(_mm(xn, otf), 0.0)

    for _ in range(2):
        xs = _mm(molwt_ref[...], xn)
        xd = _mm(molwt_ref[...], out)
        pern = _mm(_mm(attd_ref[...], xd), ottf)
        a = _leaky(_mm(atts_ref[...], xs) + pern)
        am = jnp.where(ott, jnp.broadcast_to(a, (N_GRAPHS, PADN)), -1e30)
        amax = jnp.max(am, axis=1, keepdims=True)
        amax = jnp.where(amax > -1e29, amax, 0.0)
        apern = _mm(jnp.reshape(amax, (1, N_GRAPHS)), ottf)
        ex = jnp.exp(a - apern)
        s = _mm(ex, otf)
        spern = _mm(s, ottf)
        w = ex / (spern + 1e-16)
        hg = _mm(xs * w, otf) + molb_ref[:, :1]
        hg = _elu(hg)
        out = jnp.maximum(
            _gru_t(hg, out, mwit_ref[...], mbib_ref[...],
                   mwht_ref[...], mbhb_ref[...]), 0.0)

    g = _mm(w2t_ref[...], out) + b2_ref[...]
    gr = jnp.maximum(g[0:1] * pw1a_ref[...] + g[1:2] * pw1b_ref[...]
                     + pb1_ref[...], 0.0)
    out_ref[...] = gr * pw2_ref[...] + pb2_ref[...]


def _build_tc_pool(interpret=False):
    return pl.pallas_call(
        _pool_body,
        out_shape=jax.ShapeDtypeStruct((1, N_GRAPHS), jnp.float32),
        interpret=interpret,
    )


_tc_pool = _build_tc_pool()


def kernel(x, edge_index, batch, edge_attr, W_lin1, b_lin1, gate_W1, gate_W2,
           gate_att_l, gate_att_r, gate_bias, gru0_Wi, gru0_Wh, gru0_bi,
           gru0_bh, conv_W, conv_att_src, conv_att_dst, conv_bias, grus_Wi,
           grus_Wh, grus_bi, grus_bh, mol_W, mol_att_src, mol_att_dst,
           mol_bias, molgru_Wi, molgru_Wh, molgru_bi, molgru_bh, W_lin2,
           b_lin2, pm_W1, pm_b1, pm_W2, pm_b2):
    f32 = jnp.float32
    src = edge_index[0].astype(jnp.int32)
    dst = edge_index[1].astype(jnp.int32)
    pad = PADN - N_NODES
    xT = jnp.pad(x, ((0, pad), (0, 0))).T
    batch_p = jnp.pad(batch.astype(jnp.int32), (0, pad),
                      constant_values=N_GRAPHS)

    def bcast(v, rows):
        return jnp.broadcast_to(v.astype(f32)[:, None], (rows, 128))

    x0T, valT, asrc, adst, mx = _tc_prep(
        xT, W_lin1.T, bcast(b_lin1, HID), gate_W1[:HID].T,
        bcast(gate_W1[HID], HID), gate_W2.T, gate_att_l[None, :],
        gate_att_r[None, :])
    sc_edge = _get_sc_edge()
    aggT = sc_edge(valT, asrc.reshape(PADN), adst.reshape(PADN), mx,
                   src, dst)[0]

    xcur = x0T
    L = conv_W.shape[0]
    for l in range(L + 1):
        if l == 0:
            biasc = bcast(gate_bias, HID)
            wit, wht = gru0_Wi.T, gru0_Wh.T
            bib, bhb = bcast(gru0_bi, 3 * HID), bcast(gru0_bh, 3 * HID)
        else:
            biasc = bcast(conv_bias[l - 1], HID)
            wit, wht = grus_Wi[l - 1].T, grus_Wh[l - 1].T
            bib = bcast(grus_bi[l - 1], 3 * HID)
            bhb = bcast(grus_bh[l - 1], 3 * HID)
        li = min(l, L - 1)
        xcur, xlT, asrc, adst, mx = _tc_layer(
            aggT, xcur, biasc, wit, bib, wht, bhb, conv_W[li].T,
            conv_att_src[li][None, :], conv_att_dst[li][None, :])
        if l < L:
            aggT = sc_edge(xlT, asrc.reshape(PADN), adst.reshape(PADN),
                           mx, src, dst)[0]

    def brow64(v):
        return jnp.broadcast_to(v.astype(f32)[None, None], (1, N_GRAPHS))

    g = _tc_pool(
        xcur, batch_p[None, :], batch_p[:, None], mol_W.T,
        mol_att_src[None, :], mol_att_dst[None, :], bcast(mol_bias, HID),
        molgru_Wi.T, bcast(molgru_bi, 3 * HID), molgru_Wh.T,
        bcast(molgru_bh, 3 * HID), W_lin2.T,
        jnp.broadcast_to(b_lin2.astype(f32)[:, None], (2, N_GRAPHS)),
        brow64(pm_W1[0, 0]), brow64(pm_W1[1, 0]), brow64(pm_b1[0]),
        brow64(pm_W2[0, 0]), brow64(pm_b2[0]))
    return g.reshape(N_GRAPHS, 1)


# speedup vs baseline: 1.0697x; 1.0109x over previous
import functools

import jax
import jax.numpy as jnp
from jax import lax
from jax.experimental import pallas as pl
from jax.experimental.pallas import tpu as pltpu
from jax.experimental.pallas import tpu_sc as plsc

N_NODES = 10000
N_EDGES = 320000
N_GRAPHS = 64
HID = 128
PADN = 10240
SLOPE = 0.01

_CH = 2000
_NCH = N_EDGES // _CH

_NC = 2
_NS = 16
_NW = _NC * _NS
_CPT = HID // _NW

@functools.cache
def _get_sc_edge():
    mesh = plsc.VectorSubcoreMesh(core_axis_name="c", subcore_axis_name="s",
                                  num_cores=_NC, num_subcores=_NS)
    return functools.partial(
        pl.kernel,
        mesh=mesh,
        compiler_params=pltpu.CompilerParams(needs_layout_passes=False),
        out_type=(
            jax.ShapeDtypeStruct((HID, PADN), jnp.float32),
            jax.ShapeDtypeStruct((_NC * N_EDGES,), jnp.float32),
            jax.ShapeDtypeStruct((_NC * _NS * PADN,), jnp.float32),
            jax.ShapeDtypeStruct((_NC * PADN,), jnp.float32),
        ),
        scratch_types=[
            pltpu.VMEM((_CPT, PADN), jnp.float32),
            pltpu.VMEM((_CPT, PADN), jnp.float32),
            pltpu.VMEM((PADN,), jnp.float32),
            pltpu.VMEM((PADN,), jnp.float32),
            pltpu.VMEM((PADN,), jnp.float32),
            pltpu.VMEM((2, 128), jnp.float32),
            pltpu.VMEM((_CH,), jnp.int32),
            pltpu.VMEM((_CH,), jnp.int32),
            pltpu.VMEM((_CH,), jnp.int32),
            pltpu.VMEM((_CH,), jnp.int32),
            pltpu.VMEM((_CH,), jnp.float32),
            pltpu.VMEM((_CH,), jnp.float32),
            pltpu.VMEM((_NS * 160,), jnp.float32),
            pltpu.VMEM((PADN // _NS,), jnp.float32),
            pltpu.SemaphoreType.DMA,
            pltpu.SemaphoreType.DMA,
            pltpu.SemaphoreType.DMA,
            pltpu.SemaphoreType.DMA,
            pltpu.SemaphoreType.DMA,
            pltpu.SemaphoreType.DMA,
        ],
    )(_sc_edge_body)


_EPT = N_EDGES // _NS
_NCH1 = _EPT // _CH
_NSL = PADN // _NS


def _sc_edge_body(val_hbm, asrc_hbm, adst_hbm, mx_hbm, src_hbm, dst_hbm,
                  out_hbm, ex_hbm, part_hbm, sred_hbm, xc, hc, av, bv, sv,
                  mv, sb0, db0, sb1, db1, eb0, eb1, rb, accv,
                  sem_s0, sem_d0, sem_s1, sem_d1, sem_e0, sem_e1):
    cid = lax.axis_index("c")
    sid = lax.axis_index("s")
    wid = sid * _NC + cid
    base = wid * _CPT
    bufs = ((sb0, db0, sem_s0, sem_d0, eb0, sem_e0),
            (sb1, db1, sem_s1, sem_d1, eb1, sem_e1))

    def _start_sd(b, k):
        sb, db, ss, sd, _, _ = bufs[b]
        pltpu.make_async_copy(src_hbm.at[pl.ds(k * _CH, _CH)], sb, ss).start()
        pltpu.make_async_copy(dst_hbm.at[pl.ds(k * _CH, _CH)], db, sd).start()

    def _wait_sd(b, k):
        sb, db, ss, sd, _, _ = bufs[b]
        pltpu.make_async_copy(src_hbm.at[pl.ds(k * _CH, _CH)], sb, ss).wait()
        pltpu.make_async_copy(dst_hbm.at[pl.ds(k * _CH, _CH)], db, sd).wait()

    ld0 = pltpu.make_async_copy(val_hbm.at[pl.ds(base, _CPT)], xc, sem_s0)
    ld1 = pltpu.make_async_copy(asrc_hbm, av, sem_d0)
    ld2 = pltpu.make_async_copy(adst_hbm, bv, sem_s1)
    ld3 = pltpu.make_async_copy(mx_hbm, mv, sem_d1)
    ld0.start(); ld1.start(); ld2.start(); ld3.start()

    zero16 = jnp.zeros((16,), jnp.float32)

    @plsc.parallel_loop(0, PADN // 16, unroll=8)
    def _zs(i):
        sv[pl.ds(i * 16, 16)] = zero16

    @plsc.parallel_loop(0, PADN // 16, unroll=8)
    def _zh(i):
        for c in range(_CPT):
            hc[c, pl.ds(i * 16, 16)] = zero16

    ld0.wait(); ld1.wait(); ld2.wait(); ld3.wait()
    cvec = jnp.maximum(mv[0, pl.ds(0, 16)] + mv[1, pl.ds(0, 16)], 0.0)

    ebase = sid * _EPT

    exbase = cid * N_EDGES

    def _ex_out(b, k):
        eb, se = bufs[b][4], bufs[b][5]
        return pltpu.make_async_copy(
            eb, ex_hbm.at[pl.ds(exbase + ebase + k * _CH, _CH)], se)

    def _start1(b, k):
        sb, db, ss, sd, _, _ = bufs[b]
        pltpu.make_async_copy(
            src_hbm.at[pl.ds(ebase + k * _CH, _CH)], sb, ss).start()
        pltpu.make_async_copy(
            dst_hbm.at[pl.ds(ebase + k * _CH, _CH)], db, sd).start()

    def _wait1(b, k):
        sb, db, ss, sd, _, _ = bufs[b]
        pltpu.make_async_copy(
            src_hbm.at[pl.ds(ebase + k * _CH, _CH)], sb, ss).wait()
        pltpu.make_async_copy(
            dst_hbm.at[pl.ds(ebase + k * _CH, _CH)], db, sd).wait()

    _start1(0, 0)
    for k in range(_NCH1):
        b = k % 2
        if k + 1 < _NCH1:
            _start1(1 - b, k + 1)
        _wait1(b, k)
        if k >= 2:
            _ex_out(b, k - 2).wait()
        sb, db, eb = bufs[b][0], bufs[b][1], bufs[b][4]

        @plsc.parallel_loop(0, _CH // 16, unroll=8)
        def _g1(i):
            svv = sb[pl.ds(i * 16, 16)]
            dvv = db[pl.ds(i * 16, 16)]
            t = plsc.load_gather(av, [svv]) + plsc.load_gather(bv, [dvv])
            al = jnp.maximum(t, t * SLOPE)
            ex = jnp.exp(al - cvec)
            eb[pl.ds(i * 16, 16)] = ex
            plsc.addupdate_scatter(sv, [dvv], ex)
        _ex_out(b, k).start()
    for k in (_NCH1 - 2, _NCH1 - 1):
        _ex_out(k % 2, k).wait()

    pbase = cid * _NS * PADN
    nbase = sid * _NSL
    pltpu.sync_copy(sv, part_hbm.at[pl.ds(pbase + sid * PADN, PADN)])
    plsc.subcore_barrier()

    for p in range(_NSL // 160):
        for j in range(_NS):
            pltpu.make_async_copy(
                part_hbm.at[pl.ds(pbase + j * PADN + nbase + p * 160, 160)],
                rb.at[pl.ds(j * 160, 160)], sem_e0).start()
        for j in range(_NS):
            pltpu.make_async_copy(
                part_hbm.at[pl.ds(pbase + j * PADN + nbase + p * 160, 160)],
                rb.at[pl.ds(j * 160, 160)], sem_e0).wait()

        @plsc.parallel_loop(0, 10, unroll=2)
        def _r(i):
            acc = rb[pl.ds(i * 16, 16)]
            for j in range(1, _NS):
                acc = acc + rb[pl.ds(j * 160 + i * 16, 16)]
            accv[pl.ds(p * 160 + i * 16, 16)] = acc
    pltpu.sync_copy(accv, sred_hbm.at[pl.ds(cid * PADN + nbase, _NSL)])
    plsc.subcore_barrier()
    pltpu.sync_copy(sred_hbm.at[pl.ds(cid * PADN, PADN)], sv)

    def _start2(b, k):
        _start_sd(b, k)
        eb, se = bufs[b][4], bufs[b][5]
        pltpu.make_async_copy(
            ex_hbm.at[pl.ds(exbase + k * _CH, _CH)], eb, se).start()

    def _wait2(b, k):
        _wait_sd(b, k)
        eb, se = bufs[b][4], bufs[b][5]
        pltpu.make_async_copy(
            ex_hbm.at[pl.ds(exbase + k * _CH, _CH)], eb, se).wait()

    def _proc_h(sb, db, eb):
        @plsc.parallel_loop(0, _CH // 16, unroll=4)
        def _g(i):
            svv = sb[pl.ds(i * 16, 16)]
            dvv = db[pl.ds(i * 16, 16)]
            ex = eb[pl.ds(i * 16, 16)]
            w = ex / plsc.load_gather(sv, [dvv])
            for c in range(_CPT):
                cv16 = jnp.full((16,), c, jnp.int32)
                xvv = plsc.load_gather(xc, [cv16, svv])
                plsc.addupdate_scatter(hc, [cv16, dvv], xvv * w)

    _start2(0, 0)

    def _pair(k2, _):
        c0 = 2 * k2
        _start2(1, c0 + 1)
        _wait2(0, c0)
        _proc_h(bufs[0][0], bufs[0][1], bufs[0][4])
        nxt = jnp.minimum(c0 + 2, _NCH - 1)
        _start2(0, nxt)
        _wait2(1, c0 + 1)
        _proc_h(bufs[1][0], bufs[1][1], bufs[1][4])
        return 0
    lax.fori_loop(0, _NCH // 2, _pair, 0)
    _wait2(0, _NCH - 1)

    pltpu.sync_copy(hc, out_hbm.at[pl.ds(base, _CPT)])


def _mm(a, b):
    return jnp.dot(a, b, preferred_element_type=jnp.float32)


def _leaky(v):
    return jnp.maximum(v, v * SLOPE)


def _elu(v):
    vn = jnp.minimum(v, 0.0)
    u = jnp.exp(vn)
    em1 = jnp.where(u == 1.0, vn, (u - 1.0) * vn / jnp.log(u))
    return jnp.where(v > 0, v, em1)


def _gru_t(hin, xprev, wit, bib, wht, bhb):
    gi = _mm(wit, hin) + bib[:, :1]
    gh = _mm(wht, xprev) + bhb[:, :1]
    r = jax.nn.sigmoid(gi[0:HID] + gh[0:HID])
    z = jax.nn.sigmoid(gi[HID:2 * HID] + gh[HID:2 * HID])
    n = jnp.tanh(gi[2 * HID:] + r * gh[2 * HID:])
    return (1.0 - z) * n + z * xprev


def _prep_body(xt_ref, w1t_ref, b1_ref, gw1at_ref, w1b_ref, gw2t_ref,
               attl_ref, attr_ref,
               x0_ref, val_ref, asrc_ref, adst_ref, mx_ref):
    x0 = _leaky(_mm(w1t_ref[...], xt_ref[...]) + b1_ref[:, :1])
    x0_ref[...] = x0
    ht = _leaky(_mm(gw1at_ref[...], x0) + w1b_ref[:, :1])
    val_ref[...] = _mm(gw2t_ref[...], x0)
    asrc = _mm(attl_ref[...], ht)
    adst = _mm(attr_ref[...], x0)
    asrc_ref[...] = asrc
    adst_ref[...] = adst
    m0 = jnp.broadcast_to(jnp.max(asrc), (1, 128))
    m1 = jnp.broadcast_to(jnp.max(adst), (1, 128))
    mx_ref[...] = jnp.concatenate([m0, m1], axis=0)


def _build_tc_prep(interpret=False):
    return pl.pallas_call(
        _prep_body,
        out_shape=(
            jax.ShapeDtypeStruct((HID, PADN), jnp.float32),
            jax.ShapeDtypeStruct((HID, PADN), jnp.float32),
            jax.ShapeDtypeStruct((1, PADN), jnp.float32),
            jax.ShapeDtypeStruct((1, PADN), jnp.float32),
            jax.ShapeDtypeStruct((2, 128), jnp.float32),
        ),
        interpret=interpret,
    )


_tc_prep = _build_tc_prep()


_BW = PADN // 4


def _layer_body(agg_ref, xprev_ref, biasc_ref, wit_ref, bib_ref, wht_ref,
                bhb_ref, wlt_ref, ats_ref, atd_ref,
                xn_ref, xl_ref, asrc_ref, adst_ref, mx_ref):
    hh = _elu(agg_ref[...] + biasc_ref[:, :1])
    xn = jnp.maximum(
        _gru_t(hh, xprev_ref[...], wit_ref[...], bib_ref[...],
               wht_ref[...], bhb_ref[...]), 0.0)
    xn_ref[...] = xn
    xl = _mm(wlt_ref[...], xn)
    xl_ref[...] = xl
    asrc = _mm(ats_ref[...], xl)
    adst = _mm(atd_ref[...], xl)
    asrc_ref[...] = asrc
    adst_ref[...] = adst
    cur = jnp.concatenate(
        [jnp.broadcast_to(jnp.max(asrc), (1, 128)),
         jnp.broadcast_to(jnp.max(adst), (1, 128))], axis=0)

    @pl.when(pl.program_id(0) == 0)
    def _():
        mx_ref[...] = cur

    @pl.when(pl.program_id(0) > 0)
    def _():
        mx_ref[...] = jnp.maximum(mx_ref[...], cur)


def _build_tc_layer(interpret=False):
    return pl.pallas_call(
        _layer_body,
        grid=(4,),
        interpret=interpret,
        in_specs=[
            pl.BlockSpec((HID, _BW), lambda i: (0, i)),
            pl.BlockSpec((HID, _BW), lambda i: (0, i)),
            pl.BlockSpec((HID, 128), lambda i: (0, 0)),
            pl.BlockSpec((3 * HID, HID), lambda i: (0, 0)),
            pl.BlockSpec((3 * HID, 128), lambda i: (0, 0)),
            pl.BlockSpec((3 * HID, HID), lambda i: (0, 0)),
            pl.BlockSpec((3 * HID, 128), lambda i: (0, 0)),
            pl.BlockSpec((HID, HID), lambda i: (0, 0)),
            pl.BlockSpec((1, HID), lambda i: (0, 0)),
            pl.BlockSpec((1, HID), lambda i: (0, 0)),
        ],
        out_specs=(
            pl.BlockSpec((HID, _BW), lambda i: (0, i)),
            pl.BlockSpec((HID, _BW), lambda i: (0, i)),
            pl.BlockSpec((1, _BW), lambda i: (0, i)),
            pl.BlockSpec((1, _BW), lambda i: (0, i)),
            pl.BlockSpec((2, 128), lambda i: (0, 0)),
        ),
        out_shape=(
            jax.ShapeDtypeStruct((HID, PADN), jnp.float32),
            jax.ShapeDtypeStruct((HID, PADN), jnp.float32),
            jax.ShapeDtypeStruct((1, PADN), jnp.float32),
            jax.ShapeDtypeStruct((1, PADN), jnp.float32),
            jax.ShapeDtypeStruct((2, 128), jnp.float32),
        ),
    )


_tc_layer = _build_tc_layer()


def _pool_body(xn_ref, brow_ref, bcol_ref, molwt_ref, atts_ref, attd_ref,
               molb_ref, mwit_ref, mbib_ref, mwht_ref, mbhb_ref,
               w2t_ref, b2_ref, pw1a_ref, pw1b_ref, pb1_ref, pw2_ref,
               pb2_ref, out_ref):
    xn = xn_ref[...]
    brow = brow_ref[...]
    bcol = bcol_ref[...]
    gid_r = lax.broadcasted_iota(jnp.int32, (N_GRAPHS, PADN), 0)
    ott = gid_r == brow
    ottf = ott.astype(jnp.float32)
    gid_c = lax.broadcasted_iota(jnp.int32, (PADN, N_GRAPHS), 1)
    otf = (gid_c == bcol).astype(jnp.float32)

    out = jnp.maximum(_mm(xn, otf), 0.0)

    for _ in range(2):
        xs = _mm(molwt_ref[...], xn)
        xd = _mm(molwt_ref[...], out)
        pern = _mm(_mm(attd_ref[...], xd), ottf)
        a = _leaky(_mm(atts_ref[...], xs) + pern)
        am = jnp.where(ott, jnp.broadcast_to(a, (N_GRAPHS, PADN)), -1e30)
        amax = jnp.max(am, axis=1, keepdims=True)
        amax = jnp.where(amax > -1e29, amax, 0.0)
        apern = _mm(jnp.reshape(amax, (1, N_GRAPHS)), ottf)
        ex = jnp.exp(a - apern)
        s = _mm(ex, otf)
        spern = _mm(s, ottf)
        w = ex / (spern + 1e-16)
        hg = _mm(xs * w, otf) + molb_ref[:, :1]
        hg = _elu(hg)
        out = jnp.maximum(
            _gru_t(hg, out, mwit_ref[...], mbib_ref[...],
                   mwht_ref[...], mbhb_ref[...]), 0.0)

    g = _mm(w2t_ref[...], out) + b2_ref[...]
    gr = jnp.maximum(g[0:1] * pw1a_ref[...] + g[1:2] * pw1b_ref[...]
                     + pb1_ref[...], 0.0)
    out_ref[...] = gr * pw2_ref[...] + pb2_ref[...]


def _build_tc_pool(interpret=False):
    return pl.pallas_call(
        _pool_body,
        out_shape=jax.ShapeDtypeStruct((1, N_GRAPHS), jnp.float32),
        interpret=interpret,
    )


_tc_pool = _build_tc_pool()


def kernel(x, edge_index, batch, edge_attr, W_lin1, b_lin1, gate_W1, gate_W2,
           gate_att_l, gate_att_r, gate_bias, gru0_Wi, gru0_Wh, gru0_bi,
           gru0_bh, conv_W, conv_att_src, conv_att_dst, conv_bias, grus_Wi,
           grus_Wh, grus_bi, grus_bh, mol_W, mol_att_src, mol_att_dst,
           mol_bias, molgru_Wi, molgru_Wh, molgru_bi, molgru_bh, W_lin2,
           b_lin2, pm_W1, pm_b1, pm_W2, pm_b2):
    f32 = jnp.float32
    src = edge_index[0].astype(jnp.int32)
    dst = edge_index[1].astype(jnp.int32)
    pad = PADN - N_NODES
    xT = jnp.pad(x, ((0, pad), (0, 0))).T
    batch_p = jnp.pad(batch.astype(jnp.int32), (0, pad),
                      constant_values=N_GRAPHS)

    def bcast(v, rows):
        return jnp.broadcast_to(v.astype(f32)[:, None], (rows, 128))

    x0T, valT, asrc, adst, mx = _tc_prep(
        xT, W_lin1.T, bcast(b_lin1, HID), gate_W1[:HID].T,
        bcast(gate_W1[HID], HID), gate_W2.T, gate_att_l[None, :],
        gate_att_r[None, :])
    sc_edge = _get_sc_edge()
    aggT = sc_edge(valT, asrc.reshape(PADN), adst.reshape(PADN), mx,
                   src, dst)[0]

    xcur = x0T
    L = conv_W.shape[0]
    for l in range(L + 1):
        if l == 0:
            biasc = bcast(gate_bias, HID)
            wit, wht = gru0_Wi.T, gru0_Wh.T
            bib, bhb = bcast(gru0_bi, 3 * HID), bcast(gru0_bh, 3 * HID)
        else:
            biasc = bcast(conv_bias[l - 1], HID)
            wit, wht = grus_Wi[l - 1].T, grus_Wh[l - 1].T
            bib = bcast(grus_bi[l - 1], 3 * HID)
            bhb = bcast(grus_bh[l - 1], 3 * HID)
        li = min(l, L - 1)
        xcur, xlT, asrc, adst, mx = _tc_layer(
            aggT, xcur, biasc, wit, bib, wht, bhb, conv_W[li].T,
            conv_att_src[li][None, :], conv_att_dst[li][None, :])
        if l < L:
            aggT = sc_edge(xlT, asrc.reshape(PADN), adst.reshape(PADN),
                           mx, src, dst)[0]

    def brow64(v):
        return jnp.broadcast_to(v.astype(f32)[None, None], (1, N_GRAPHS))

    g = _tc_pool(
        xcur, batch_p[None, :], batch_p[:, None], mol_W.T,
        mol_att_src[None, :], mol_att_dst[None, :], bcast(mol_bias, HID),
        molgru_Wi.T, bcast(molgru_bi, 3 * HID), molgru_Wh.T,
        bcast(molgru_bh, 3 * HID), W_lin2.T,
        jnp.broadcast_to(b_lin2.astype(f32)[:, None], (2, N_GRAPHS)),
        brow64(pm_W1[0, 0]), brow64(pm_W1[1, 0]), brow64(pm_b1[0]),
        brow64(pm_W2[0, 0]), brow64(pm_b2[0]))
    return g.reshape(N_GRAPHS, 1)
